# Initial kernel scaffold; baseline (speedup 1.0000x reference)
import jax
import jax.numpy as jnp
from jax.experimental import pallas as pl


def kernel(x, edge_index, edge_attr, W_l, b_l, W_r, b_r, W_e, att, bias_attn, W1, b1, W2, b2, g0, be0, g1, be1, g2, be2):
    raise NotImplementedError("write your pallas kernel here")


# speedup vs baseline: 3.0861x; 3.0861x over previous
import jax
import jax.numpy as jnp
import numpy as np
from jax import lax
from jax.experimental import pallas as pl
from jax.experimental.pallas import tpu as pltpu
from jax.experimental.pallas import tpu_sc as plsc

N = 10000
E = 160000
D = 256
H = 8
DE = 16
DFF = 2048
HD = H * D
HALF = 1024
EN = E + N
ENP = 172032
NP = 10240
NP8 = NP // 8
EP8 = ENP // 8
C = 16
NW = 32
PER_W = ENP // NW
PER_T = ENP // 16
E_PER_W0 = E // 16

_mesh = plsc.VectorSubcoreMesh(core_axis_name="c", subcore_axis_name="s")
_params = pltpu.CompilerParams(needs_layout_passes=False)


def _perm_cols():
    idx = np.arange(HD).reshape(H, D)
    return np.concatenate([idx[:, :128].reshape(-1), idx[:, 128:].reshape(-1)])


def _layer_norm(x, g, b, eps=1e-5):
    mu = jnp.mean(x, axis=-1, keepdims=True)
    var = jnp.mean((x - mu) ** 2, axis=-1, keepdims=True)
    return (x - mu) / jnp.sqrt(var + eps) * g + b


def _zero16(buf, nrow):
    zvec = jnp.zeros((16,), jnp.float32)

    def row(i, _):
        for q in range(8):
            buf[i, pl.ds(q * 16, 16)] = zvec
        return 0
    lax.fori_loop(0, nrow, row, 0)


def _k0_body(didx_hbm, eattr_hbm, acc_att_hbm, acc_cnt_hbm,
             didx_v, drow_v, dl_v, eav, eav128, cnt128, zb, att_sh, cnt_sh):
    c = lax.axis_index("c")
    s = lax.axis_index("s")
    lane = lax.iota(jnp.int32, 16)
    cvec = jnp.where(lane == 0, 1.0, 0.0).astype(jnp.float32)

    @pl.when(c == 0)
    def _():
        _zero16(zb, 16)

        def zrow(i, _):
            pltpu.sync_copy(zb, att_sh.at[pl.ds(s * 80 + i * 16, 16), :])
            pltpu.sync_copy(zb, cnt_sh.at[pl.ds(s * 80 + i * 16, 16), :])
            return 0
        lax.fori_loop(0, 5, zrow, 0)

    plsc.subcore_barrier()

    @pl.when(c == 0)
    def _():
        def chunk(ci, _):
            base = s * E_PER_W0 + ci * C
            pltpu.sync_copy(didx_hbm.at[pl.ds(base, C)], didx_v)
            pltpu.sync_copy(eattr_hbm.at[pl.ds(base, C), :], eav)
            dv = didx_v[...]
            drow_v[...] = lax.shift_right_logical(dv, 3)
            dl_v[...] = (dv & 7) * 16
            _zero16(eav128, 16)
            _zero16(cnt128, 16)

            def edge(j, _):
                jf = jnp.broadcast_to(j, (16,)).astype(jnp.int32)
                sp = plsc.load_gather(dl_v, [jf])
                plsc.store_scatter(eav128, [jf, sp + lane], eav[j, :])
                plsc.store_scatter(cnt128, [jf, sp + lane], cvec)
                return 0
            lax.fori_loop(0, C, edge, 0)

            pltpu.sync_copy(eav128, att_sh.at[drow_v], add=True)
            pltpu.sync_copy(cnt128, cnt_sh.at[drow_v], add=True)
            return 0
        lax.fori_loop(0, E_PER_W0 // C, chunk, 0)

    plsc.subcore_barrier()

    @pl.when(c == 0)
    def _():
        pltpu.sync_copy(att_sh.at[pl.ds(s * 80, 80), :],
                        acc_att_hbm.at[pl.ds(s * 80, 80), :])
        pltpu.sync_copy(cnt_sh.at[pl.ds(s * 80, 80), :],
                        acc_cnt_hbm.at[pl.ds(s * 80, 80), :])


_k0 = pl.kernel(
    _k0_body, mesh=_mesh, compiler_params=_params,
    out_type=[jax.ShapeDtypeStruct((NP8, 128), jnp.float32),
              jax.ShapeDtypeStruct((NP8, 128), jnp.float32)],
    scratch_types=[pltpu.VMEM((C,), jnp.int32),
                   pltpu.VMEM((C,), jnp.int32),
                   pltpu.VMEM((C,), jnp.int32),
                   pltpu.VMEM((C, 16), jnp.float32),
                   pltpu.VMEM((C, 128), jnp.float32),
                   pltpu.VMEM((C, 128), jnp.float32),
                   pltpu.VMEM((16, 128), jnp.float32),
                   pltpu.VMEM_SHARED((NP8, 128), jnp.float32),
                   pltpu.VMEM_SHARED((NP8, 128), jnp.float32)],
)


def _k1_body(x_ref, wla_ref, wlb_ref, wra_ref, wrb_ref, bla_ref, blb_ref,
             bra_ref, brb_ref, g0_ref, be0_ref, acca_ref, accc_ref,
             src_ref, xla_ref, xlb_ref, xra_ref, xrb_ref, la_ref):
    src = _layer_norm(x_ref[...], g0_ref[...], be0_ref[...])
    src_ref[...] = src
    xla_ref[...] = jnp.dot(src, wla_ref[...], preferred_element_type=jnp.float32) + bla_ref[...]
    xlb_ref[...] = jnp.dot(src, wlb_ref[...], preferred_element_type=jnp.float32) + blb_ref[...]
    xra_ref[...] = jnp.dot(src, wra_ref[...], preferred_element_type=jnp.float32) + bra_ref[...]
    xrb_ref[...] = jnp.dot(src, wrb_ref[...], preferred_element_type=jnp.float32) + brb_ref[...]
    la_ref[...] = acca_ref[...] / jnp.maximum(accc_ref[...][:, 0:1], 1.0)


def _k1(x, wla, wlb, wra, wrb, bla, blb, bra, brb, g0, be0, acca, accc):
    R = 400
    row = lambda i: (i, 0)
    fix = lambda i: (0, 0)
    return pl.pallas_call(
        _k1_body,
        grid=(N // R,),
        in_specs=[pl.BlockSpec((R, D), row),
                  pl.BlockSpec((D, HALF), fix), pl.BlockSpec((D, HALF), fix),
                  pl.BlockSpec((D, HALF), fix), pl.BlockSpec((D, HALF), fix),
                  pl.BlockSpec((1, HALF), fix), pl.BlockSpec((1, HALF), fix),
                  pl.BlockSpec((1, HALF), fix), pl.BlockSpec((1, HALF), fix),
                  pl.BlockSpec((1, D), fix), pl.BlockSpec((1, D), fix),
                  pl.BlockSpec((R, 16), row), pl.BlockSpec((R, 16), row)],
        out_specs=[pl.BlockSpec((R, D), row),
                   pl.BlockSpec((R, HALF), row), pl.BlockSpec((R, HALF), row),
                   pl.BlockSpec((R, HALF), row), pl.BlockSpec((R, HALF), row),
                   pl.BlockSpec((R, 16), row)],
        out_shape=[jax.ShapeDtypeStruct((N, D), jnp.float32),
                   jax.ShapeDtypeStruct((N, HALF), jnp.float32),
                   jax.ShapeDtypeStruct((N, HALF), jnp.float32),
                   jax.ShapeDtypeStruct((N, HALF), jnp.float32),
                   jax.ShapeDtypeStruct((N, HALF), jnp.float32),
                   jax.ShapeDtypeStruct((N, 16), jnp.float32)],
    )(x, wla, wlb, wra, wrb, bla, blb, bra, brb, g0, be0, acca, accc)


def _k2_body(ea_ref, we_ref, e_ref):
    e_ref[...] = jnp.dot(ea_ref[...], we_ref[...], preferred_element_type=jnp.float32)


def _k2(ea, wep):
    R = 512
    return pl.pallas_call(
        _k2_body,
        grid=(ENP // R,),
        in_specs=[pl.BlockSpec((R, DE), lambda i: (i, 0)),
                  pl.BlockSpec((DE, HD), lambda i: (0, 0))],
        out_specs=pl.BlockSpec((R, HD), lambda i: (i, 0)),
        out_shape=jax.ShapeDtypeStruct((ENP, HD), jnp.float32),
    )(ea, wep)


def _k3_body(sidx_hbm, didxg_hbm, didxs_hbm, xla_hbm, xlb_hbm, xra_hbm,
             xrb_hbm, e_hbm, attp_hbm, logits_hbm, den_hbm,
             sidx_v, didxg_v, didxs_v, drow_v, dl_v, xla, xlb, xra, xrb,
             ebuf, attv, lg, wb, zb, tbuf, den_sh, sem0, sem1, sem2, sem3):
    c = lax.axis_index("c")
    s = lax.axis_index("s")
    wid = s * 2 + c
    lane = lax.iota(jnp.int32, 16)
    zvec = jnp.zeros((16,), jnp.float32)

    pltpu.sync_copy(attp_hbm, attv)
    _zero16(zb, 16)

    def zrow(i, _):
        pltpu.sync_copy(zb, den_sh.at[pl.ds(s * 80 + i * 16, 16), :])
        return 0
    lax.fori_loop(0, 5, zrow, 0)
    plsc.subcore_barrier()

    def chunk(ci, _):
        base = wid * PER_W + ci * C
        pltpu.sync_copy(sidx_hbm.at[pl.ds(base, C)], sidx_v)
        pltpu.sync_copy(didxg_hbm.at[pl.ds(base, C)], didxg_v)
        pltpu.sync_copy(didxs_hbm.at[pl.ds(base, C)], didxs_v)
        cp0 = pltpu.async_copy(xla_hbm.at[sidx_v], xla, sem0)
        cp1 = pltpu.async_copy(xlb_hbm.at[sidx_v], xlb, sem1)
        cp2 = pltpu.async_copy(xra_hbm.at[didxg_v], xra, sem2)
        cp3 = pltpu.async_copy(xrb_hbm.at[didxg_v], xrb, sem3)
        pltpu.sync_copy(e_hbm.at[pl.ds(base, C), :], ebuf)
        dv = didxs_v[...]
        drow_v[...] = lax.shift_right_logical(dv, 3)
        dl_v[...] = (dv & 7) * 16
        _zero16(wb, 16)
        cp0.wait(); cp1.wait(); cp2.wait(); cp3.wait()

        def edge(j, _):
            def head(h, _h):
                def blk(q, acc):
                    col = h * 128 + q * 16
                    m = xla[j, pl.ds(col, 16)] + xra[j, pl.ds(col, 16)] + ebuf[j, pl.ds(col, 16)]
                    m = jnp.where(m > 0, m, 0.2 * m)
                    acc = acc + m * attv[pl.ds(col, 16)]
                    colb = HALF + col
                    m2 = xlb[j, pl.ds(col, 16)] + xrb[j, pl.ds(col, 16)] + ebuf[j, pl.ds(colb, 16)]
                    m2 = jnp.where(m2 > 0, m2, 0.2 * m2)
                    return acc + m2 * attv[pl.ds(colb, 16)]
                tbuf[h, :] = lax.fori_loop(0, 8, blk, zvec)
                return 0
            lax.fori_loop(0, H, head, 0)
            lvec = zvec
            for k in range(16):
                lvec = lvec + plsc.load_gather(tbuf, [lane, jnp.full((16,), k, jnp.int32)])
            lvec = jnp.where(lane < H, lvec, 0.0)
            jrow = lax.shift_right_logical(j, 3)
            jcol = (j & 7) * 16
            lg[jrow, pl.ds(jcol, 16)] = lvec
            wv = jnp.where(lane < H, jnp.exp(lvec), 0.0)
            jf = jnp.broadcast_to(j, (16,)).astype(jnp.int32)
            sp = plsc.load_gather(dl_v, [jf])
            plsc.store_scatter(wb, [jf, sp + lane], wv)
            return 0
        lax.fori_loop(0, C, edge, 0)

        pltpu.sync_copy(lg, logits_hbm.at[pl.ds(wid * (PER_W // 8) + ci * 2, 2), :])
        pltpu.sync_copy(wb, den_sh.at[drow_v], add=True)
        return 0

    lax.fori_loop(0, PER_W // C, chunk, 0)
    plsc.subcore_barrier()

    pltpu.sync_copy(den_sh.at[pl.ds(s * 80, 80), :],
                    den_hbm.at[pl.ds(c * NP8 + s * 80, 80), :])


_k3 = pl.kernel(
    _k3_body, mesh=_mesh, compiler_params=_params,
    out_type=[jax.ShapeDtypeStruct((EP8, 128), jnp.float32),
              jax.ShapeDtypeStruct((2 * NP8, 128), jnp.float32)],
    scratch_types=[pltpu.VMEM((C,), jnp.int32),
                   pltpu.VMEM((C,), jnp.int32),
                   pltpu.VMEM((C,), jnp.int32),
                   pltpu.VMEM((C,), jnp.int32),
                   pltpu.VMEM((C,), jnp.int32),
                   pltpu.VMEM((C, HALF), jnp.float32),
                   pltpu.VMEM((C, HALF), jnp.float32),
                   pltpu.VMEM((C, HALF), jnp.float32),
                   pltpu.VMEM((C, HALF), jnp.float32),
                   pltpu.VMEM((C, HD), jnp.float32),
                   pltpu.VMEM((HD,), jnp.float32),
                   pltpu.VMEM((2, 128), jnp.float32),
                   pltpu.VMEM((C, 128), jnp.float32),
                   pltpu.VMEM((16, 128), jnp.float32),
                   pltpu.VMEM((16, 16), jnp.float32),
                   pltpu.VMEM_SHARED((NP8, 128), jnp.float32),
                   pltpu.SemaphoreType.DMA,
                   pltpu.SemaphoreType.DMA,
                   pltpu.SemaphoreType.DMA,
                   pltpu.SemaphoreType.DMA],
)


def _k4_body(sidx_hbm, didxg_hbm, didxs_hbm, logits_hbm, den0_hbm, den1_hbm,
             xlab_hbm, alpha_hbm, pages_hbm,
             sidx_v, didxg_v, didxs_v, drow_v, dl_v, lg, d0b, d1b, xbuf,
             fold, ab, zb, acc_sh, sem0, sem1, sem2):
    c = lax.axis_index("c")
    s = lax.axis_index("s")
    lane = lax.iota(jnp.int32, 16)
    zvec = jnp.zeros((16,), jnp.float32)

    _zero16(zb, 16)

    def zrow(i, _):
        pltpu.sync_copy(zb, acc_sh.at[pl.ds(s * 640 + i * 16, 16), :])
        return 0
    lax.fori_loop(0, 40, zrow, 0)
    plsc.subcore_barrier()

    def chunk(ci, _):
        base = s * PER_T + ci * C
        pltpu.sync_copy(sidx_hbm.at[pl.ds(base, C)], sidx_v)
        pltpu.sync_copy(didxg_hbm.at[pl.ds(base, C)], didxg_v)
        pltpu.sync_copy(didxs_hbm.at[pl.ds(base, C)], didxs_v)
        dv = didxg_v[...]
        drow_v[...] = lax.shift_right_logical(dv, 3)
        dl_v[...] = (dv & 7) * 16
        sidx_v[...] = sidx_v[...] + jnp.broadcast_to(c * N, (16,)).astype(jnp.int32)
        cp0 = pltpu.async_copy(den0_hbm.at[drow_v], d0b, sem0)
        cp1 = pltpu.async_copy(den1_hbm.at[drow_v], d1b, sem1)
        cp2 = pltpu.async_copy(xlab_hbm.at[sidx_v], xbuf, sem2)
        pltpu.sync_copy(logits_hbm.at[pl.ds(s * (PER_T // 8) + ci * 2, 2), :], lg)
        cp2.wait()
        cp0.wait(); cp1.wait()

        def edge(j, _):
            jrow = lax.shift_right_logical(j, 3)
            jcol = (j & 7) * 16
            jf = jnp.broadcast_to(j, (16,)).astype(jnp.int32)
            sp = plsc.load_gather(dl_v, [jf])
            lv = lg[jrow, pl.ds(jcol, 16)]
            w = jnp.where(lane < H, jnp.exp(lv), 0.0)
            g0 = plsc.load_gather(d0b, [jf, sp + lane])
            g1 = plsc.load_gather(d1b, [jf, sp + lane])
            dtot = jnp.where(lane < H, g0 + g1, 1.0)
            ab[jrow, pl.ds(jcol, 16)] = w / dtot
            return 0
        lax.fori_loop(0, C, edge, 0)

        @pl.when(c == 0)
        def _():
            pltpu.sync_copy(ab, alpha_hbm.at[pl.ds(s * (PER_T // 8) + ci * 2, 2), :])

        def edge2(j, _):
            jrow = lax.shift_right_logical(j, 3)
            jcol = (j & 7) * 16

            def head(h, accs):
                aspl = plsc.load_gather(
                    ab, [jnp.broadcast_to(jrow, (16,)).astype(jnp.int32),
                         jnp.broadcast_to(jcol + h, (16,)).astype(jnp.int32)]) * (1.0 / H)
                return tuple(accs[q] + aspl * xbuf[j, pl.ds(h * 128 + q * 16, 16)]
                             for q in range(8))
            accs = lax.fori_loop(0, H, head, (zvec,) * 8)
            for q in range(8):
                fold[j, pl.ds(q * 16, 16)] = accs[q]
            return 0
        lax.fori_loop(0, C, edge2, 0)

        pltpu.sync_copy(fold, acc_sh.at[didxs_v], add=True)
        return 0

    lax.fori_loop(0, PER_T // C, chunk, 0)
    plsc.subcore_barrier()

    pltpu.sync_copy(acc_sh.at[pl.ds(s * 640, 640), :],
                    pages_hbm.at[pl.ds(c * NP + s * 640, 640), :])


_k4 = pl.kernel(
    _k4_body, mesh=_mesh, compiler_params=_params,
    out_type=[jax.ShapeDtypeStruct((EP8, 128), jnp.float32),
              jax.ShapeDtypeStruct((2 * NP, 128), jnp.float32)],
    scratch_types=[pltpu.VMEM((C,), jnp.int32),
                   pltpu.VMEM((C,), jnp.int32),
                   pltpu.VMEM((C,), jnp.int32),
                   pltpu.VMEM((C,), jnp.int32),
                   pltpu.VMEM((C,), jnp.int32),
                   pltpu.VMEM((2, 128), jnp.float32),
                   pltpu.VMEM((C, 128), jnp.float32),
                   pltpu.VMEM((C, 128), jnp.float32),
                   pltpu.VMEM((C, HALF), jnp.float32),
                   pltpu.VMEM((C, 128), jnp.float32),
                   pltpu.VMEM((2, 128), jnp.float32),
                   pltpu.VMEM((16, 128), jnp.float32),
                   pltpu.VMEM_SHARED((NP, 128), jnp.float32),
                   pltpu.SemaphoreType.DMA,
                   pltpu.SemaphoreType.DMA,
                   pltpu.SemaphoreType.DMA],
)


def _k5_body(src_ref, pa_ref, pb_ref, bias_ref, g1_ref, be1_ref, w1_ref,
             b1_ref, w2_ref, b2_ref, g2_ref, be2_ref, out_ref):
    attn = jnp.concatenate([pa_ref[...], pb_ref[...]], axis=1) + bias_ref[...]
    h = src_ref[...] + attn
    h = _layer_norm(h, g1_ref[...], be1_ref[...])
    z = jnp.dot(h, w1_ref[...], preferred_element_type=jnp.float32) + b1_ref[...]
    z = 0.5 * z * (1.0 + lax.erf(z * np.float32(1.0 / np.sqrt(2.0))))
    ff = jnp.dot(z, w2_ref[...], preferred_element_type=jnp.float32) + b2_ref[...]
    out_ref[...] = _layer_norm(h + ff, g2_ref[...], be2_ref[...])


def _k5(src, pa, pb, bias, g1, be1, w1, b1, w2, b2, g2, be2):
    R = 400
    row = lambda i: (i, 0)
    fix = lambda i: (0, 0)
    return pl.pallas_call(
        _k5_body,
        grid=(N // R,),
        in_specs=[pl.BlockSpec((R, D), row),
                  pl.BlockSpec((R, 128), row), pl.BlockSpec((R, 128), row),
                  pl.BlockSpec((1, D), fix),
                  pl.BlockSpec((1, D), fix), pl.BlockSpec((1, D), fix),
                  pl.BlockSpec((D, DFF), fix), pl.BlockSpec((1, DFF), fix),
                  pl.BlockSpec((DFF, D), fix), pl.BlockSpec((1, D), fix),
                  pl.BlockSpec((1, D), fix), pl.BlockSpec((1, D), fix)],
        out_specs=pl.BlockSpec((R, D), row),
        out_shape=jax.ShapeDtypeStruct((N, D), jnp.float32),
    )(src, pa, pb, bias, g1, be1, w1, b1, w2, b2, g2, be2)


def kernel(x, edge_index, edge_attr, W_l, b_l, W_r, b_r, W_e, att, bias_attn,
           W1, b1, W2, b2, g0, be0, g1, be1, g2, be2):
    perm = jnp.asarray(_perm_cols())
    wlp = W_l[:, perm]
    wrp = W_r[:, perm]
    wep = W_e[:, perm]
    blp = b_l[perm]
    brp = b_r[perm]
    attp = att.reshape(-1)[perm]

    d_e = edge_index[1].astype(jnp.int32)

    acca, accc = _k0(d_e, edge_attr)
    acca = acca.reshape(NP, 16)
    accc = accc.reshape(NP, 16)

    src, xla, xlb, xra, xrb, la = _k1(
        x, wlp[:, :HALF], wlp[:, HALF:], wrp[:, :HALF], wrp[:, HALF:],
        blp[:HALF].reshape(1, HALF), blp[HALF:].reshape(1, HALF),
        brp[:HALF].reshape(1, HALF), brp[HALF:].reshape(1, HALF),
        g0.reshape(1, D), be0.reshape(1, D), acca[:N], accc[:N])

    ea = jnp.concatenate([edge_attr, la, jnp.zeros((ENP - EN, DE), jnp.float32)], axis=0)
    e = _k2(ea, wep)

    pad = ENP - EN
    arange_n = jnp.arange(N, dtype=jnp.int32)
    s_idx = jnp.concatenate([edge_index[0].astype(jnp.int32), arange_n,
                             jnp.zeros((pad,), jnp.int32)])
    d_gat = jnp.concatenate([d_e, arange_n, jnp.zeros((pad,), jnp.int32)])
    d_sca = jnp.concatenate([d_e, arange_n, jnp.full((pad,), N, jnp.int32)])

    logits, den = _k3(s_idx, d_gat, d_sca, xla, xlb, xra, xrb, e, attp)

    xlab = jnp.concatenate([xla, xlb], axis=0)
    alpha_p, pages = _k4(s_idx, d_gat, d_sca, logits, den[:NP8], den[NP8:], xlab)

    outp = _k5(src, pages[:N], pages[NP:NP + N], bias_attn.reshape(1, D),
               g1.reshape(1, D), be1.reshape(1, D), W1, b1.reshape(1, DFF),
               W2, b2.reshape(1, D), g2.reshape(1, D), be2.reshape(1, D))

    alpha = alpha_p.reshape(ENP, 16)[:EN, :H]
    return outp, alpha


# speedup vs baseline: 3.4751x; 1.1260x over previous
import jax
import jax.numpy as jnp
import numpy as np
from jax import lax
from jax.experimental import pallas as pl
from jax.experimental.pallas import tpu as pltpu
from jax.experimental.pallas import tpu_sc as plsc

N = 10000
E = 160000
D = 256
H = 8
DE = 16
DFF = 2048
HD = H * D
HALF = 1024
EN = E + N
ENP = 172032
NP = 10240
NP8 = NP // 8
EP8 = ENP // 8
C = 16
NW = 32
PER_W = ENP // NW
PER_T = ENP // 16
E_PER_W0 = E // 16
C0 = 80
C4 = 32
LG4 = C4 * 16 // 128

_mesh = plsc.VectorSubcoreMesh(core_axis_name="c", subcore_axis_name="s")
_params = pltpu.CompilerParams(needs_layout_passes=False)


def _perm_cols():
    idx = np.arange(HD).reshape(H, D)
    return np.concatenate([idx[:, :128].reshape(-1), idx[:, 128:].reshape(-1)])


def _layer_norm(x, g, b, eps=1e-5):
    mu = jnp.mean(x, axis=-1, keepdims=True)
    var = jnp.mean((x - mu) ** 2, axis=-1, keepdims=True)
    return (x - mu) / jnp.sqrt(var + eps) * g + b


def _zero16(buf, nrow):
    zvec = jnp.zeros((16,), jnp.float32)

    def row(i, _):
        for q in range(8):
            buf[i, pl.ds(q * 16, 16)] = zvec
        return 0
    lax.fori_loop(0, nrow, row, 0)


def _k0_body(didx_hbm, eattr_hbm, acc_att_hbm, acc_cnt_hbm,
             didx_v, drow_v, dl_v, eav, eav128, cnt128, zb, att_sh, cnt_sh):
    c = lax.axis_index("c")
    s = lax.axis_index("s")
    lane = lax.iota(jnp.int32, 16)
    cvec = jnp.where(lane == 0, 1.0, 0.0).astype(jnp.float32)

    @pl.when(c == 0)
    def _():
        _zero16(zb, 16)

        def zrow(i, _):
            pltpu.sync_copy(zb, att_sh.at[pl.ds(s * 80 + i * 16, 16), :])
            pltpu.sync_copy(zb, cnt_sh.at[pl.ds(s * 80 + i * 16, 16), :])
            return 0
        lax.fori_loop(0, 5, zrow, 0)

    plsc.subcore_barrier()

    @pl.when(c == 0)
    def _():
        def chunk(ci, _):
            base = s * E_PER_W0 + ci * C0
            pltpu.sync_copy(didx_hbm.at[pl.ds(base, C0)], didx_v)
            pltpu.sync_copy(eattr_hbm.at[pl.ds(base, C0), :], eav)
            def idxseg(i, _):
                dv = didx_v[pl.ds(i * 16, 16)]
                drow_v[pl.ds(i * 16, 16)] = lax.shift_right_logical(dv, 3)
                dl_v[pl.ds(i * 16, 16)] = (dv & 7) * 16
                return 0
            lax.fori_loop(0, C0 // 16, idxseg, 0)
            _zero16(eav128, C0)
            _zero16(cnt128, C0)

            def edge(j, _):
                jf = jnp.broadcast_to(j, (16,)).astype(jnp.int32)
                sp = plsc.load_gather(dl_v, [jf])
                plsc.store_scatter(eav128, [jf, sp + lane], eav[j, :])
                plsc.store_scatter(cnt128, [jf, sp + lane], cvec)
                return 0
            lax.fori_loop(0, C0, edge, 0)

            pltpu.sync_copy(eav128, att_sh.at[drow_v], add=True)
            pltpu.sync_copy(cnt128, cnt_sh.at[drow_v], add=True)
            return 0
        lax.fori_loop(0, E_PER_W0 // C0, chunk, 0)

    plsc.subcore_barrier()

    @pl.when(c == 0)
    def _():
        pltpu.sync_copy(att_sh.at[pl.ds(s * 80, 80), :],
                        acc_att_hbm.at[pl.ds(s * 80, 80), :])
        pltpu.sync_copy(cnt_sh.at[pl.ds(s * 80, 80), :],
                        acc_cnt_hbm.at[pl.ds(s * 80, 80), :])


_k0 = pl.kernel(
    _k0_body, mesh=_mesh, compiler_params=_params,
    out_type=[jax.ShapeDtypeStruct((NP8, 128), jnp.float32),
              jax.ShapeDtypeStruct((NP8, 128), jnp.float32)],
    scratch_types=[pltpu.VMEM((C0,), jnp.int32),
                   pltpu.VMEM((C0,), jnp.int32),
                   pltpu.VMEM((C0,), jnp.int32),
                   pltpu.VMEM((C0, 16), jnp.float32),
                   pltpu.VMEM((C0, 128), jnp.float32),
                   pltpu.VMEM((C0, 128), jnp.float32),
                   pltpu.VMEM((16, 128), jnp.float32),
                   pltpu.VMEM_SHARED((NP8, 128), jnp.float32),
                   pltpu.VMEM_SHARED((NP8, 128), jnp.float32)],
)


def _k1_body(x_ref, wla_ref, wlb_ref, wra_ref, wrb_ref, bla_ref, blb_ref,
             bra_ref, brb_ref, g0_ref, be0_ref, acca_ref, accc_ref,
             src_ref, xla_ref, xlb_ref, xra_ref, xrb_ref, la_ref):
    src = _layer_norm(x_ref[...], g0_ref[...], be0_ref[...])
    src_ref[...] = src
    xla_ref[...] = jnp.dot(src, wla_ref[...], preferred_element_type=jnp.float32) + bla_ref[...]
    xlb_ref[...] = jnp.dot(src, wlb_ref[...], preferred_element_type=jnp.float32) + blb_ref[...]
    xra_ref[...] = jnp.dot(src, wra_ref[...], preferred_element_type=jnp.float32) + bra_ref[...]
    xrb_ref[...] = jnp.dot(src, wrb_ref[...], preferred_element_type=jnp.float32) + brb_ref[...]
    la_ref[...] = acca_ref[...] / jnp.maximum(accc_ref[...][:, 0:1], 1.0)


def _k1(x, wla, wlb, wra, wrb, bla, blb, bra, brb, g0, be0, acca, accc):
    R = 400
    row = lambda i: (i, 0)
    fix = lambda i: (0, 0)
    return pl.pallas_call(
        _k1_body,
        grid=(N // R,),
        in_specs=[pl.BlockSpec((R, D), row),
                  pl.BlockSpec((D, HALF), fix), pl.BlockSpec((D, HALF), fix),
                  pl.BlockSpec((D, HALF), fix), pl.BlockSpec((D, HALF), fix),
                  pl.BlockSpec((1, HALF), fix), pl.BlockSpec((1, HALF), fix),
                  pl.BlockSpec((1, HALF), fix), pl.BlockSpec((1, HALF), fix),
                  pl.BlockSpec((1, D), fix), pl.BlockSpec((1, D), fix),
                  pl.BlockSpec((R, 16), row), pl.BlockSpec((R, 16), row)],
        out_specs=[pl.BlockSpec((R, D), row),
                   pl.BlockSpec((R, HALF), row), pl.BlockSpec((R, HALF), row),
                   pl.BlockSpec((R, HALF), row), pl.BlockSpec((R, HALF), row),
                   pl.BlockSpec((R, 16), row)],
        out_shape=[jax.ShapeDtypeStruct((N, D), jnp.float32),
                   jax.ShapeDtypeStruct((N, HALF), jnp.float32),
                   jax.ShapeDtypeStruct((N, HALF), jnp.float32),
                   jax.ShapeDtypeStruct((N, HALF), jnp.float32),
                   jax.ShapeDtypeStruct((N, HALF), jnp.float32),
                   jax.ShapeDtypeStruct((N, 16), jnp.float32)],
    )(x, wla, wlb, wra, wrb, bla, blb, bra, brb, g0, be0, acca, accc)


def _k2_body(ea_ref, we_ref, e_ref):
    e_ref[...] = jnp.dot(ea_ref[...], we_ref[...], preferred_element_type=jnp.float32)


def _k2(ea, wep):
    R = 512
    return pl.pallas_call(
        _k2_body,
        grid=(ENP // R,),
        in_specs=[pl.BlockSpec((R, DE), lambda i: (i, 0)),
                  pl.BlockSpec((DE, HD), lambda i: (0, 0))],
        out_specs=pl.BlockSpec((R, HD), lambda i: (i, 0)),
        out_shape=jax.ShapeDtypeStruct((ENP, HD), jnp.float32),
    )(ea, wep)


def _k3_body(sidx_hbm, didxg_hbm, didxs_hbm, xla_hbm, xlb_hbm, xra_hbm,
             xrb_hbm, e_hbm, attp_hbm, logits_hbm, den_hbm,
             sidx_v, didxg_v, didxs_v, drow_v, dl_v, xla, xlb, xra, xrb,
             ebuf, attv, lg, wb, zb, tbuf, den_sh, sem0, sem1, sem2, sem3):
    c = lax.axis_index("c")
    s = lax.axis_index("s")
    wid = s * 2 + c
    lane = lax.iota(jnp.int32, 16)
    zvec = jnp.zeros((16,), jnp.float32)

    pltpu.sync_copy(attp_hbm, attv)
    _zero16(zb, 16)

    def zrow(i, _):
        pltpu.sync_copy(zb, den_sh.at[pl.ds(s * 80 + i * 16, 16), :])
        return 0
    lax.fori_loop(0, 5, zrow, 0)
    plsc.subcore_barrier()

    def chunk(ci, _):
        base = wid * PER_W + ci * C
        pltpu.sync_copy(sidx_hbm.at[pl.ds(base, C)], sidx_v)
        pltpu.sync_copy(didxg_hbm.at[pl.ds(base, C)], didxg_v)
        pltpu.sync_copy(didxs_hbm.at[pl.ds(base, C)], didxs_v)
        cp0 = pltpu.async_copy(xla_hbm.at[sidx_v], xla, sem0)
        cp1 = pltpu.async_copy(xlb_hbm.at[sidx_v], xlb, sem1)
        cp2 = pltpu.async_copy(xra_hbm.at[didxg_v], xra, sem2)
        cp3 = pltpu.async_copy(xrb_hbm.at[didxg_v], xrb, sem3)
        pltpu.sync_copy(e_hbm.at[pl.ds(base, C), :], ebuf)
        dv = didxs_v[...]
        drow_v[...] = lax.shift_right_logical(dv, 3)
        dl_v[...] = (dv & 7) * 16
        _zero16(wb, 16)
        cp0.wait(); cp1.wait(); cp2.wait(); cp3.wait()

        def edge(j, _):
            def head(h, _h):
                def blk(q, acc):
                    col = h * 128 + q * 16
                    m = xla[j, pl.ds(col, 16)] + xra[j, pl.ds(col, 16)] + ebuf[j, pl.ds(col, 16)]
                    m = jnp.where(m > 0, m, 0.2 * m)
                    acc = acc + m * attv[pl.ds(col, 16)]
                    colb = HALF + col
                    m2 = xlb[j, pl.ds(col, 16)] + xrb[j, pl.ds(col, 16)] + ebuf[j, pl.ds(colb, 16)]
                    m2 = jnp.where(m2 > 0, m2, 0.2 * m2)
                    return acc + m2 * attv[pl.ds(colb, 16)]
                tbuf[h, :] = lax.fori_loop(0, 8, blk, zvec)
                return 0
            lax.fori_loop(0, H, head, 0)
            lvec = zvec
            for k in range(16):
                lvec = lvec + plsc.load_gather(tbuf, [lane, jnp.full((16,), k, jnp.int32)])
            lvec = jnp.where(lane < H, lvec, 0.0)
            jrow = lax.shift_right_logical(j, 3)
            jcol = (j & 7) * 16
            lg[jrow, pl.ds(jcol, 16)] = lvec
            wv = jnp.where(lane < H, jnp.exp(lvec), 0.0)
            jf = jnp.broadcast_to(j, (16,)).astype(jnp.int32)
            sp = plsc.load_gather(dl_v, [jf])
            plsc.store_scatter(wb, [jf, sp + lane], wv)
            return 0
        lax.fori_loop(0, C, edge, 0)

        pltpu.sync_copy(lg, logits_hbm.at[pl.ds(wid * (PER_W // 8) + ci * 2, 2), :])
        pltpu.sync_copy(wb, den_sh.at[drow_v], add=True)
        return 0

    lax.fori_loop(0, PER_W // C, chunk, 0)
    plsc.subcore_barrier()

    pltpu.sync_copy(den_sh.at[pl.ds(s * 80, 80), :],
                    den_hbm.at[pl.ds(c * NP8 + s * 80, 80), :])


_k3 = pl.kernel(
    _k3_body, mesh=_mesh, compiler_params=_params,
    out_type=[jax.ShapeDtypeStruct((EP8, 128), jnp.float32),
              jax.ShapeDtypeStruct((2 * NP8, 128), jnp.float32)],
    scratch_types=[pltpu.VMEM((C,), jnp.int32),
                   pltpu.VMEM((C,), jnp.int32),
                   pltpu.VMEM((C,), jnp.int32),
                   pltpu.VMEM((C,), jnp.int32),
                   pltpu.VMEM((C,), jnp.int32),
                   pltpu.VMEM((C, HALF), jnp.float32),
                   pltpu.VMEM((C, HALF), jnp.float32),
                   pltpu.VMEM((C, HALF), jnp.float32),
                   pltpu.VMEM((C, HALF), jnp.float32),
                   pltpu.VMEM((C, HD), jnp.float32),
                   pltpu.VMEM((HD,), jnp.float32),
                   pltpu.VMEM((2, 128), jnp.float32),
                   pltpu.VMEM((C, 128), jnp.float32),
                   pltpu.VMEM((16, 128), jnp.float32),
                   pltpu.VMEM((16, 16), jnp.float32),
                   pltpu.VMEM_SHARED((NP8, 128), jnp.float32),
                   pltpu.SemaphoreType.DMA,
                   pltpu.SemaphoreType.DMA,
                   pltpu.SemaphoreType.DMA,
                   pltpu.SemaphoreType.DMA],
)


def _k4_body(sidx_hbm, didxg_hbm, didxs_hbm, logits_hbm, den0_hbm, den1_hbm,
             xlab_hbm, alpha_hbm, pages_hbm,
             sidx_v, didxg_v, didxs_v, drow_v, dl_v, lg, d0b, d1b, xbuf,
             fold, ab, zb, acc_sh, sem0, sem1, sem2):
    c = lax.axis_index("c")
    s = lax.axis_index("s")
    lane = lax.iota(jnp.int32, 16)
    zvec = jnp.zeros((16,), jnp.float32)

    _zero16(zb, 16)

    def zrow(i, _):
        pltpu.sync_copy(zb, acc_sh.at[pl.ds(s * 640 + i * 16, 16), :])
        return 0
    lax.fori_loop(0, 40, zrow, 0)
    plsc.subcore_barrier()

    def chunk(ci, _):
        base = s * PER_T + ci * C4
        pltpu.sync_copy(sidx_hbm.at[pl.ds(base, C4)], sidx_v)
        pltpu.sync_copy(didxg_hbm.at[pl.ds(base, C4)], didxg_v)
        pltpu.sync_copy(didxs_hbm.at[pl.ds(base, C4)], didxs_v)
        coff = jnp.broadcast_to(c * N, (16,)).astype(jnp.int32)

        def idxseg(i, _):
            dv = didxg_v[pl.ds(i * 16, 16)]
            drow_v[pl.ds(i * 16, 16)] = lax.shift_right_logical(dv, 3)
            dl_v[pl.ds(i * 16, 16)] = (dv & 7) * 16
            sidx_v[pl.ds(i * 16, 16)] = sidx_v[pl.ds(i * 16, 16)] + coff
            return 0
        lax.fori_loop(0, C4 // 16, idxseg, 0)
        cp0 = pltpu.async_copy(den0_hbm.at[drow_v], d0b, sem0)
        cp1 = pltpu.async_copy(den1_hbm.at[drow_v], d1b, sem1)
        cp2 = pltpu.async_copy(xlab_hbm.at[sidx_v], xbuf, sem2)
        pltpu.sync_copy(logits_hbm.at[pl.ds(s * (PER_T // 8) + ci * LG4, LG4), :], lg)
        cp2.wait()
        cp0.wait(); cp1.wait()

        def edge(j, _):
            jrow = lax.shift_right_logical(j, 3)
            jcol = (j & 7) * 16
            jf = jnp.broadcast_to(j, (16,)).astype(jnp.int32)
            sp = plsc.load_gather(dl_v, [jf])
            lv = lg[jrow, pl.ds(jcol, 16)]
            w = jnp.where(lane < H, jnp.exp(lv), 0.0)
            g0 = plsc.load_gather(d0b, [jf, sp + lane])
            g1 = plsc.load_gather(d1b, [jf, sp + lane])
            dtot = jnp.where(lane < H, g0 + g1, 1.0)
            ab[jrow, pl.ds(jcol, 16)] = w / dtot
            return 0
        lax.fori_loop(0, C4, edge, 0)

        @pl.when(c == 0)
        def _():
            pltpu.sync_copy(ab, alpha_hbm.at[pl.ds(s * (PER_T // 8) + ci * LG4, LG4), :])

        def edge2(j, _):
            jrow = lax.shift_right_logical(j, 3)
            jcol = (j & 7) * 16

            def head(h, accs):
                aspl = plsc.load_gather(
                    ab, [jnp.broadcast_to(jrow, (16,)).astype(jnp.int32),
                         jnp.broadcast_to(jcol + h, (16,)).astype(jnp.int32)]) * (1.0 / H)
                return tuple(accs[q] + aspl * xbuf[j, pl.ds(h * 128 + q * 16, 16)]
                             for q in range(8))
            accs = lax.fori_loop(0, H, head, (zvec,) * 8)
            for q in range(8):
                fold[j, pl.ds(q * 16, 16)] = accs[q]
            return 0
        lax.fori_loop(0, C4, edge2, 0)

        pltpu.sync_copy(fold, acc_sh.at[didxs_v], add=True)
        return 0

    lax.fori_loop(0, PER_T // C4, chunk, 0)
    plsc.subcore_barrier()

    pltpu.sync_copy(acc_sh.at[pl.ds(s * 640, 640), :],
                    pages_hbm.at[pl.ds(c * NP + s * 640, 640), :])


_k4 = pl.kernel(
    _k4_body, mesh=_mesh, compiler_params=_params,
    out_type=[jax.ShapeDtypeStruct((EP8, 128), jnp.float32),
              jax.ShapeDtypeStruct((2 * NP, 128), jnp.float32)],
    scratch_types=[pltpu.VMEM((C4,), jnp.int32),
                   pltpu.VMEM((C4,), jnp.int32),
                   pltpu.VMEM((C4,), jnp.int32),
                   pltpu.VMEM((C4,), jnp.int32),
                   pltpu.VMEM((C4,), jnp.int32),
                   pltpu.VMEM((LG4, 128), jnp.float32),
                   pltpu.VMEM((C4, 128), jnp.float32),
                   pltpu.VMEM((C4, 128), jnp.float32),
                   pltpu.VMEM((C4, HALF), jnp.float32),
                   pltpu.VMEM((C4, 128), jnp.float32),
                   pltpu.VMEM((LG4, 128), jnp.float32),
                   pltpu.VMEM((16, 128), jnp.float32),
                   pltpu.VMEM_SHARED((NP, 128), jnp.float32),
                   pltpu.SemaphoreType.DMA,
                   pltpu.SemaphoreType.DMA,
                   pltpu.SemaphoreType.DMA],
)


def _k5_body(src_ref, pa_ref, pb_ref, bias_ref, g1_ref, be1_ref, w1_ref,
             b1_ref, w2_ref, b2_ref, g2_ref, be2_ref, out_ref):
    attn = jnp.concatenate([pa_ref[...], pb_ref[...]], axis=1) + bias_ref[...]
    h = src_ref[...] + attn
    h = _layer_norm(h, g1_ref[...], be1_ref[...])
    z = jnp.dot(h, w1_ref[...], preferred_element_type=jnp.float32) + b1_ref[...]
    z = 0.5 * z * (1.0 + lax.erf(z * np.float32(1.0 / np.sqrt(2.0))))
    ff = jnp.dot(z, w2_ref[...], preferred_element_type=jnp.float32) + b2_ref[...]
    out_ref[...] = _layer_norm(h + ff, g2_ref[...], be2_ref[...])


def _k5(src, pa, pb, bias, g1, be1, w1, b1, w2, b2, g2, be2):
    R = 400
    row = lambda i: (i, 0)
    fix = lambda i: (0, 0)
    return pl.pallas_call(
        _k5_body,
        grid=(N // R,),
        in_specs=[pl.BlockSpec((R, D), row),
                  pl.BlockSpec((R, 128), row), pl.BlockSpec((R, 128), row),
                  pl.BlockSpec((1, D), fix),
                  pl.BlockSpec((1, D), fix), pl.BlockSpec((1, D), fix),
                  pl.BlockSpec((D, DFF), fix), pl.BlockSpec((1, DFF), fix),
                  pl.BlockSpec((DFF, D), fix), pl.BlockSpec((1, D), fix),
                  pl.BlockSpec((1, D), fix), pl.BlockSpec((1, D), fix)],
        out_specs=pl.BlockSpec((R, D), row),
        out_shape=jax.ShapeDtypeStruct((N, D), jnp.float32),
    )(src, pa, pb, bias, g1, be1, w1, b1, w2, b2, g2, be2)


def kernel(x, edge_index, edge_attr, W_l, b_l, W_r, b_r, W_e, att, bias_attn,
           W1, b1, W2, b2, g0, be0, g1, be1, g2, be2):
    perm = jnp.asarray(_perm_cols())
    wlp = W_l[:, perm]
    wrp = W_r[:, perm]
    wep = W_e[:, perm]
    blp = b_l[perm]
    brp = b_r[perm]
    attp = att.reshape(-1)[perm]

    d_e = edge_index[1].astype(jnp.int32)

    acca, accc = _k0(d_e, edge_attr)
    acca = acca.reshape(NP, 16)
    accc = accc.reshape(NP, 16)

    src, xla, xlb, xra, xrb, la = _k1(
        x, wlp[:, :HALF], wlp[:, HALF:], wrp[:, :HALF], wrp[:, HALF:],
        blp[:HALF].reshape(1, HALF), blp[HALF:].reshape(1, HALF),
        brp[:HALF].reshape(1, HALF), brp[HALF:].reshape(1, HALF),
        g0.reshape(1, D), be0.reshape(1, D), acca[:N], accc[:N])

    ea = jnp.concatenate([edge_attr, la, jnp.zeros((ENP - EN, DE), jnp.float32)], axis=0)
    e = _k2(ea, wep)

    pad = ENP - EN
    arange_n = jnp.arange(N, dtype=jnp.int32)
    s_idx = jnp.concatenate([edge_index[0].astype(jnp.int32), arange_n,
                             jnp.zeros((pad,), jnp.int32)])
    d_gat = jnp.concatenate([d_e, arange_n, jnp.zeros((pad,), jnp.int32)])
    d_sca = jnp.concatenate([d_e, arange_n, jnp.full((pad,), N, jnp.int32)])

    logits, den = _k3(s_idx, d_gat, d_sca, xla, xlb, xra, xrb, e, attp)

    xlab = jnp.concatenate([xla, xlb], axis=0)
    alpha_p, pages = _k4(s_idx, d_gat, d_sca, logits, den[:NP8], den[NP8:], xlab)

    outp = _k5(src, pages[:N], pages[NP:NP + N], bias_attn.reshape(1, D),
               g1.reshape(1, D), be1.reshape(1, D), W1, b1.reshape(1, DFF),
               W2, b2.reshape(1, D), g2.reshape(1, D), be2.reshape(1, D))

    alpha = alpha_p.reshape(ENP, 16)[:EN, :H]
    return outp, alpha
